# Initial kernel scaffold; baseline (speedup 1.0000x reference)
import jax
import jax.numpy as jnp
from jax.experimental import pallas as pl


def kernel(atom_features, r, angle_h, edge_index, lg_edge_index, W_atom, b_atom, W_e1, b_e1, W_e2, b_e2, W_a1, b_a1, W_a2, b_a2, egc_W, egc_b, W_fc, b_fc):
    raise NotImplementedError("write your pallas kernel here")


# speedup vs baseline: 1.3155x; 1.3155x over previous
import functools

import numpy as np
import jax
import jax.numpy as jnp
from jax.experimental import pallas as pl
from jax.experimental.pallas import tpu as pltpu

HID = 64
BN_EPS = 1e-5
SEG_EPS = 1e-6


def _row_block(n, cap=2048):
    r = 8
    for c in range(8, cap + 1, 8):
        if n % c == 0:
            r = c
    return r


def _grid_call(kern, n, ins, in_widths, out_widths, n_stats, row_block=None):
    R = row_block or _row_block(n)
    grid = n // R
    in_specs = []
    for a, w in zip(ins, in_widths):
        if w is None:
            in_specs.append(pl.BlockSpec(a.shape, lambda i: (0,) * a.ndim))
        else:
            in_specs.append(pl.BlockSpec((R, w), lambda i: (i, 0)))
    out_specs = [pl.BlockSpec((R, w), lambda i: (i, 0)) for w in out_widths]
    out_shape = [jax.ShapeDtypeStruct((n, w), jnp.float32) for w in out_widths]
    for _ in range(n_stats):
        out_specs.append(pl.BlockSpec((2, HID), lambda i: (0, 0)))
        out_shape.append(jax.ShapeDtypeStruct((2, HID), jnp.float32))
    outs = pl.pallas_call(
        functools.partial(kern, grid=grid),
        grid=(grid,),
        in_specs=in_specs,
        out_specs=out_specs,
        out_shape=out_shape,
        scratch_shapes=[pltpu.VMEM((2, HID), jnp.float32)] * n_stats,
    )(*ins)
    return outs


def _accum_stats(t, i, grid, s_ref, acc_ref):
    ps = jnp.concatenate(
        [jnp.sum(t, axis=0, keepdims=True),
         jnp.sum(t * t, axis=0, keepdims=True)], axis=0)

    @pl.when(i == 0)
    def _():
        acc_ref[...] = ps

    @pl.when(i > 0)
    def _():
        acc_ref[...] = acc_ref[...] + ps

    @pl.when(i == grid - 1)
    def _():
        s_ref[...] = acc_ref[...]


def _bn_apply(t, s, n):
    mu = s[0:1, :] / n
    var = s[1:2, :] / n - mu * mu
    return (t - mu) * jax.lax.rsqrt(var + BN_EPS)


def _silu(v):
    return v * jax.nn.sigmoid(v)


def _mm_stats(x, W, b):
    n = x.shape[0]

    def kern(x_ref, w_ref, b_ref, t_ref, s_ref, acc_ref, *, grid):
        i = pl.program_id(0)
        t = jnp.dot(x_ref[...], w_ref[...],
                    preferred_element_type=jnp.float32) + b_ref[...]
        t_ref[...] = t
        _accum_stats(t, i, grid, s_ref, acc_ref)

    t, s = _grid_call(kern, n, [x, W, b.reshape(1, -1)],
                      [x.shape[1], None, None], [HID], 1)
    return t, s


def _rbf_mm_stats(d2col, W, b, vmin, vmax, bins, is_r):
    n = d2col.shape[0]
    centers = jnp.asarray(
        np.linspace(vmin, vmax, bins, dtype=np.float32)).reshape(1, bins)
    gamma = 1.0 / float(np.diff(np.linspace(vmin, vmax, bins)).mean())

    def kern(d_ref, c_ref, w_ref, b_ref, t_ref, s_ref, acc_ref, *, grid):
        i = pl.program_id(0)
        db = d_ref[...]
        if is_r:
            db = jnp.sqrt(jnp.sum(db * db, axis=1, keepdims=True))
        rbf = jnp.exp(-gamma * (db - c_ref[...]) ** 2)
        t = jnp.dot(rbf, w_ref[...],
                    preferred_element_type=jnp.float32) + b_ref[...]
        t_ref[...] = t
        _accum_stats(t, i, grid, s_ref, acc_ref)

    t, s = _grid_call(kern, n, [d2col, centers, W, b.reshape(1, -1)],
                      [d2col.shape[1], None, None, None], [HID], 1)
    return t, s


def _bnsilu_mm_stats(t1, s1, W, b):
    n = t1.shape[0]

    def kern(t1_ref, s1_ref, w_ref, b_ref, t_ref, s_ref, acc_ref, *, grid):
        i = pl.program_id(0)
        u = _silu(_bn_apply(t1_ref[...], s1_ref[...], n))
        t = jnp.dot(u, w_ref[...],
                    preferred_element_type=jnp.float32) + b_ref[...]
        t_ref[...] = t
        _accum_stats(t, i, grid, s_ref, acc_ref)

    t, s = _grid_call(kern, n, [t1, s1, W, b.reshape(1, -1)],
                      [HID, None, None, None], [HID], 1)
    return t, s


def _bnsilu(t, s):
    n = t.shape[0]

    def kern(t_ref, s_ref, o_ref, *, grid):
        o_ref[...] = _silu(_bn_apply(t_ref[...], s_ref[...], n))

    (o,) = _grid_call(kern, n, [t, s], [HID, None], [HID], 0)
    return o


def _residual_bnsilu(t, s, res):
    n = t.shape[0]

    def kern(t_ref, s_ref, r_ref, o_ref, *, grid):
        o_ref[...] = r_ref[...] + _silu(_bn_apply(t_ref[...], s_ref[...], n))

    (o,) = _grid_call(kern, n, [t, s, res], [HID, None, HID], [HID], 0)
    return o


def _mm3(x, Wcat, bcat):
    n = x.shape[0]

    def kern(x_ref, w_ref, b_ref, o1_ref, o2_ref, o3_ref, *, grid):
        t = jnp.dot(x_ref[...], w_ref[...],
                    preferred_element_type=jnp.float32) + b_ref[...]
        o1_ref[...] = t[:, :2 * HID]
        o2_ref[...] = t[:, 2 * HID:3 * HID]
        o3_ref[...] = t[:, 3 * HID:]

    o1, o2, o3 = _grid_call(kern, n, [x, Wcat, bcat.reshape(1, -1)],
                            [HID, None, None], [2 * HID, HID, HID], 0)
    return o1, o2, o3


def _mm(x, W, b):
    n = x.shape[0]

    def kern(x_ref, w_ref, b_ref, o_ref, *, grid):
        o_ref[...] = jnp.dot(x_ref[...], w_ref[...],
                             preferred_element_type=jnp.float32) + b_ref[...]

    (o,) = _grid_call(kern, n, [x, W, b.reshape(1, -1)],
                      [HID, None, None], [HID], 0)
    return o


def _edge_ew(G1, G2, yW2):
    E = G1.shape[0]

    def kern(g1_ref, g2_ref, y_ref, m_ref, sp_ref, s_ref, acc_ref, *, grid):
        i = pl.program_id(0)
        g1 = g1_ref[...]
        m = g1[:, :HID] + g2_ref[...] + y_ref[...]
        sig = jax.nn.sigmoid(m)
        m_ref[...] = m
        sp_ref[:, :HID] = sig
        sp_ref[:, HID:] = g1[:, HID:] * sig
        _accum_stats(m, i, grid, s_ref, acc_ref)

    m, sp, s = _grid_call(kern, E, [G1, G2, yW2],
                          [2 * HID, HID, HID], [HID, 2 * HID], 1)
    return m, sp, s


def _add_div_stats(xW3, S0S1):
    n = xW3.shape[0]

    def kern(x_ref, ss_ref, t_ref, s_ref, acc_ref, *, grid):
        i = pl.program_id(0)
        ss = ss_ref[...]
        t = x_ref[...] + ss[:, HID:] / (ss[:, :HID] + SEG_EPS)
        t_ref[...] = t
        _accum_stats(t, i, grid, s_ref, acc_ref)

    t, s = _grid_call(kern, n, [xW3, S0S1], [HID, 2 * HID], [HID], 1)
    return t, s


def _readout(x, W_fc, b_fc):
    n = x.shape[0]
    R = _row_block(n)
    grid = n // R

    def kern(x_ref, w_ref, b_ref, o_ref, acc_ref):
        i = pl.program_id(0)
        ps = jnp.sum(x_ref[...], axis=0, keepdims=True)

        @pl.when(i == 0)
        def _():
            acc_ref[...] = ps

        @pl.when(i > 0)
        def _():
            acc_ref[...] = acc_ref[...] + ps

        @pl.when(i == grid - 1)
        def _():
            h = acc_ref[...] / n
            o_ref[...] = jnp.dot(h, w_ref[...],
                                 preferred_element_type=jnp.float32) + b_ref[...]

    out = pl.pallas_call(
        kern,
        grid=(grid,),
        in_specs=[pl.BlockSpec((R, HID), lambda i: (i, 0)),
                  pl.BlockSpec((HID, 1), lambda i: (0, 0)),
                  pl.BlockSpec((1, 1), lambda i: (0, 0))],
        out_specs=pl.BlockSpec((1, 1), lambda i: (0, 0)),
        out_shape=jax.ShapeDtypeStruct((1, 1), jnp.float32),
        scratch_shapes=[pltpu.VMEM((1, HID), jnp.float32)],
    )(x, W_fc, b_fc.reshape(1, 1))
    return jnp.squeeze(out)


def _gather_rows(T_sg, T_d, src, dst):
    G1 = jnp.take(T_sg, src, axis=0)
    G2 = jnp.take(T_d, dst, axis=0)
    return G1, G2


def _segment_sum(SP, dst, n):
    return jax.ops.segment_sum(SP, dst, num_segments=n)


def _egc_layer(x, y, W, b, src, dst, n):
    Wcat = jnp.concatenate([W[0], W[4], W[1], W[3]], axis=1)
    bcat = jnp.concatenate([b[0], b[4], b[1], b[3]])
    T_sg, T_d, xW3 = _mm3(x, Wcat, bcat)
    yW2 = _mm(y, W[2], b[2])
    G1, G2 = _gather_rows(T_sg, T_d, src, dst)
    m, SP, stats_m = _edge_ew(G1, G2, yW2)
    S0S1 = _segment_sum(SP, dst, n)
    t, stats_t = _add_div_stats(xW3, S0S1)
    x_new = _residual_bnsilu(t, stats_t, x)
    y_new = _residual_bnsilu(m, stats_m, y)
    return x_new, y_new


def kernel(atom_features, r, angle_h, edge_index, lg_edge_index, W_atom,
           b_atom, W_e1, b_e1, W_e2, b_e2, W_a1, b_a1, W_a2, b_a2, egc_W,
           egc_b, W_fc, b_fc):
    src, dst = edge_index[0], edge_index[1]
    lsrc, ldst = lg_edge_index[0], lg_edge_index[1]
    N = atom_features.shape[0]
    E = r.shape[0]

    t, s = _rbf_mm_stats(angle_h.reshape(-1, 1), W_a1, b_a1, -1.0, 1.0, 40,
                         is_r=False)
    t, s = _bnsilu_mm_stats(t, s, W_a2, b_a2)
    z = _bnsilu(t, s)

    t, s = _mm_stats(atom_features, W_atom, b_atom)
    x = _bnsilu(t, s)

    t, s = _rbf_mm_stats(r, W_e1, b_e1, 0.0, 8.0, 80, is_r=True)
    t, s = _bnsilu_mm_stats(t, s, W_e2, b_e2)
    y = _bnsilu(t, s)

    x, m = _egc_layer(x, y, egc_W[0], egc_b[0], src, dst, N)
    y, z = _egc_layer(m, z, egc_W[1], egc_b[1], lsrc, ldst, E)
    x, m = _egc_layer(x, y, egc_W[2], egc_b[2], src, dst, N)
    y, z = _egc_layer(m, z, egc_W[3], egc_b[3], lsrc, ldst, E)
    x, y = _egc_layer(x, y, egc_W[4], egc_b[4], src, dst, N)
    x, y = _egc_layer(x, y, egc_W[5], egc_b[5], src, dst, N)

    return _readout(x, W_fc, b_fc)


# speedup vs baseline: 2.2447x; 1.7063x over previous
import functools

import numpy as np
import jax
import jax.numpy as jnp
from jax import lax
from jax.experimental import pallas as pl
from jax.experimental.pallas import tpu as pltpu
from jax.experimental.pallas import tpu_sc as plsc

SC_NC = 2
SC_NS = 16
SC_NW = SC_NC * SC_NS

HID = 64
BN_EPS = 1e-5
SEG_EPS = 1e-6


def _row_block(n, cap=2048):
    r = 8
    for c in range(8, cap + 1, 8):
        if n % c == 0:
            r = c
    return r


def _grid_call(kern, n, ins, in_widths, out_widths, n_stats, row_block=None):
    R = row_block or _row_block(n)
    grid = n // R
    in_specs = []
    for a, w in zip(ins, in_widths):
        if w is None:
            in_specs.append(
                pl.BlockSpec(a.shape, lambda i, nd=a.ndim: (0,) * nd))
        else:
            in_specs.append(pl.BlockSpec((R, w), lambda i: (i, 0)))
    out_specs = [pl.BlockSpec((R, w), lambda i: (i, 0)) for w in out_widths]
    out_shape = [jax.ShapeDtypeStruct((n, w), jnp.float32) for w in out_widths]
    for _ in range(n_stats):
        out_specs.append(pl.BlockSpec((2, HID), lambda i: (0, 0)))
        out_shape.append(jax.ShapeDtypeStruct((2, HID), jnp.float32))
    outs = pl.pallas_call(
        functools.partial(kern, grid=grid),
        grid=(grid,),
        in_specs=in_specs,
        out_specs=out_specs,
        out_shape=out_shape,
        scratch_shapes=[pltpu.VMEM((2, HID), jnp.float32)] * n_stats,
    )(*ins)
    return outs


def _accum_stats(t, i, grid, s_ref, acc_ref):
    ps = jnp.concatenate(
        [jnp.sum(t, axis=0, keepdims=True),
         jnp.sum(t * t, axis=0, keepdims=True)], axis=0)

    @pl.when(i == 0)
    def _():
        acc_ref[...] = ps

    @pl.when(i > 0)
    def _():
        acc_ref[...] = acc_ref[...] + ps

    @pl.when(i == grid - 1)
    def _():
        s_ref[...] = acc_ref[...]


def _bn_apply(t, s, n):
    mu = s[0:1, :] / n
    var = s[1:2, :] / n - mu * mu
    return (t - mu) * jax.lax.rsqrt(var + BN_EPS)


def _silu(v):
    return v * jax.nn.sigmoid(v)


def _mm_stats(x, W, b):
    n = x.shape[0]

    def kern(x_ref, w_ref, b_ref, t_ref, s_ref, acc_ref, *, grid):
        i = pl.program_id(0)
        t = jnp.dot(x_ref[...], w_ref[...],
                    preferred_element_type=jnp.float32) + b_ref[...]
        t_ref[...] = t
        _accum_stats(t, i, grid, s_ref, acc_ref)

    t, s = _grid_call(kern, n, [x, W, b.reshape(1, -1)],
                      [x.shape[1], None, None], [HID], 1)
    return t, s


def _rbf_mm_stats(d2col, W, b, vmin, vmax, bins, is_r):
    n = d2col.shape[0]
    centers = jnp.asarray(
        np.linspace(vmin, vmax, bins, dtype=np.float32)).reshape(1, bins)
    gamma = 1.0 / float(np.diff(np.linspace(vmin, vmax, bins)).mean())

    def kern(d_ref, c_ref, w_ref, b_ref, t_ref, s_ref, acc_ref, *, grid):
        i = pl.program_id(0)
        db = d_ref[...]
        if is_r:
            db = jnp.sqrt(jnp.sum(db * db, axis=1, keepdims=True))
        rbf = jnp.exp(-gamma * (db - c_ref[...]) ** 2)
        t = jnp.dot(rbf, w_ref[...],
                    preferred_element_type=jnp.float32) + b_ref[...]
        t_ref[...] = t
        _accum_stats(t, i, grid, s_ref, acc_ref)

    t, s = _grid_call(kern, n, [d2col, centers, W, b.reshape(1, -1)],
                      [d2col.shape[1], None, None, None], [HID], 1)
    return t, s


def _bnsilu_mm_stats(t1, s1, W, b):
    n = t1.shape[0]

    def kern(t1_ref, s1_ref, w_ref, b_ref, t_ref, s_ref, acc_ref, *, grid):
        i = pl.program_id(0)
        u = _silu(_bn_apply(t1_ref[...], s1_ref[...], n))
        t = jnp.dot(u, w_ref[...],
                    preferred_element_type=jnp.float32) + b_ref[...]
        t_ref[...] = t
        _accum_stats(t, i, grid, s_ref, acc_ref)

    t, s = _grid_call(kern, n, [t1, s1, W, b.reshape(1, -1)],
                      [HID, None, None, None], [HID], 1)
    return t, s


def _bnsilu(t, s):
    n = t.shape[0]

    def kern(t_ref, s_ref, o_ref, *, grid):
        o_ref[...] = _silu(_bn_apply(t_ref[...], s_ref[...], n))

    (o,) = _grid_call(kern, n, [t, s], [HID, None], [HID], 0)
    return o


def _residual_bnsilu(t, s, res):
    n = t.shape[0]

    def kern(t_ref, s_ref, r_ref, o_ref, *, grid):
        o_ref[...] = r_ref[...] + _silu(_bn_apply(t_ref[...], s_ref[...], n))

    (o,) = _grid_call(kern, n, [t, s, res], [HID, None, HID], [HID], 0)
    return o


def _mm3(x, Wcat, bcat):
    n = x.shape[0]

    def kern(x_ref, w_ref, b_ref, o1_ref, o2_ref, *, grid):
        t = jnp.dot(x_ref[...], w_ref[...],
                    preferred_element_type=jnp.float32) + b_ref[...]
        o1_ref[...] = t[:, :2 * HID]
        o2_ref[...] = t[:, 2 * HID:]

    o1, o2 = _grid_call(kern, n, [x, Wcat, bcat.reshape(1, -1)],
                        [HID, None, None], [2 * HID, 2 * HID], 0)
    return o1, o2


def _mm(x, W, b):
    n = x.shape[0]

    def kern(x_ref, w_ref, b_ref, o_ref, *, grid):
        o_ref[...] = jnp.dot(x_ref[...], w_ref[...],
                             preferred_element_type=jnp.float32) + b_ref[...]

    (o,) = _grid_call(kern, n, [x, W, b.reshape(1, -1)],
                      [HID, None, None], [HID], 0)
    return o


def _edge_ew(G1, G2, yW2):
    E = G1.shape[0]

    def kern(g1_ref, g2_ref, y_ref, m_ref, sp_ref, s_ref, acc_ref, *, grid):
        i = pl.program_id(0)
        g1 = g1_ref[...]
        m = g1[:, :HID] + g2_ref[:, :HID] + y_ref[...]
        sig = jax.nn.sigmoid(m)
        m_ref[...] = m
        sp_ref[:, :HID] = sig
        sp_ref[:, HID:] = g1[:, HID:] * sig
        _accum_stats(m, i, grid, s_ref, acc_ref)

    m, sp, s = _grid_call(kern, E, [G1, G2, yW2],
                          [2 * HID, 2 * HID, HID], [HID, 2 * HID], 1)
    return m, sp, s


def _add_div_stats(T_dx, parts):
    n = T_dx.shape[0]

    def kern(x_ref, *refs, grid):
        part_refs = refs[:len(parts)]
        t_ref, s_ref, acc_ref = refs[len(parts):]
        i = pl.program_id(0)
        ss = part_refs[0][...]
        for pr in part_refs[1:]:
            ss = ss + pr[...]
        t = x_ref[:, HID:] + ss[:, HID:] / (ss[:, :HID] + SEG_EPS)
        t_ref[...] = t
        _accum_stats(t, i, grid, s_ref, acc_ref)

    t, s = _grid_call(kern, n, [T_dx] + list(parts),
                      [2 * HID] + [2 * HID] * len(parts), [HID], 1)
    return t, s


def _readout(x, W_fc, b_fc):
    n = x.shape[0]
    R = _row_block(n)
    grid = n // R

    def kern(x_ref, w_ref, b_ref, o_ref, acc_ref):
        i = pl.program_id(0)
        ps = jnp.sum(x_ref[...], axis=0, keepdims=True)

        @pl.when(i == 0)
        def _():
            acc_ref[...] = ps

        @pl.when(i > 0)
        def _():
            acc_ref[...] = acc_ref[...] + ps

        @pl.when(i == grid - 1)
        def _():
            h = acc_ref[...] / n
            o_ref[...] = jnp.dot(h, w_ref[...],
                                 preferred_element_type=jnp.float32) + b_ref[...]

    out = pl.pallas_call(
        kern,
        grid=(grid,),
        in_specs=[pl.BlockSpec((R, HID), lambda i: (i, 0)),
                  pl.BlockSpec((HID, 1), lambda i: (0, 0)),
                  pl.BlockSpec((1, 1), lambda i: (0, 0))],
        out_specs=pl.BlockSpec((1, 1), lambda i: (0, 0)),
        out_shape=jax.ShapeDtypeStruct((1, 1), jnp.float32),
        scratch_shapes=[pltpu.VMEM((1, HID), jnp.float32)],
    )(x, W_fc, b_fc.reshape(1, 1))
    return jnp.squeeze(out)


def _gather_rows(T_sg, T_d, src, dst):
    E = src.shape[0]
    C = 128
    nchunks = E // C
    base_per_w = nchunks // SC_NW
    extra = nchunks - base_per_w * SC_NW
    mesh = plsc.VectorSubcoreMesh(core_axis_name="c", subcore_axis_name="s")

    @functools.partial(
        pl.kernel, mesh=mesh,
        out_type=[jax.ShapeDtypeStruct((E, 2 * HID), jnp.float32),
                  jax.ShapeDtypeStruct((E, 2 * HID), jnp.float32)],
        scratch_types=[pltpu.VMEM((C,), jnp.int32),
                       pltpu.VMEM((C,), jnp.int32),
                       pltpu.VMEM((C, 2 * HID), jnp.float32),
                       pltpu.VMEM((C, 2 * HID), jnp.float32),
                       pltpu.SemaphoreType.DMA,
                       pltpu.SemaphoreType.DMA],
    )
    def k(tsg_hbm, td_hbm, src_hbm, dst_hbm, g1_hbm, g2_hbm,
          idx1, idx2, rows1, rows2, sem1, sem2):
        wid = lax.axis_index("s") * SC_NC + lax.axis_index("c")
        nj = base_per_w + jnp.where(wid < extra, 1, 0)

        @pl.loop(0, nj)
        def _chunk(j):
            off = pl.multiple_of((wid + j * SC_NW) * C, C)
            pltpu.sync_copy(src_hbm.at[pl.ds(off, C)], idx1)
            pltpu.sync_copy(dst_hbm.at[pl.ds(off, C)], idx2)
            cp1 = pltpu.async_copy(tsg_hbm.at[idx1], rows1, sem1)
            cp2 = pltpu.async_copy(td_hbm.at[idx2], rows2, sem2)
            cp1.wait()
            cp2.wait()
            pltpu.sync_copy(rows1, g1_hbm.at[pl.ds(off, C)])
            pltpu.sync_copy(rows2, g2_hbm.at[pl.ds(off, C)])

    return k(T_sg, T_d, src, dst)


def _segment_sum(SP, dst, n):
    return jax.ops.segment_sum(SP, dst, num_segments=n)


def _sc_scatter_node(SP, dst, n):
    E = SP.shape[0]
    C = 128
    assert E % C == 0
    nchunks = E // C
    n_pad = -(-n // (SC_NS * 8)) * (SC_NS * 8)
    stripe = n_pad // SC_NS
    zeros = jnp.zeros((stripe, 2 * HID), jnp.float32)
    mesh = plsc.VectorSubcoreMesh(core_axis_name="c", subcore_axis_name="s")

    @functools.partial(
        pl.kernel, mesh=mesh,
        out_type=jax.ShapeDtypeStruct((SC_NC, n_pad, 2 * HID), jnp.float32),
        scratch_types=[pltpu.VMEM((C,), jnp.int32),
                       pltpu.VMEM((C, 2 * HID), jnp.float32),
                       pltpu.VMEM_SHARED((n_pad, 2 * HID), jnp.float32),
                       pltpu.SemaphoreType.DMA],
    )
    def k(sp_hbm, dst_hbm, z_hbm, out_hbm, idx, rows, acc, sem):
        c = lax.axis_index("c")
        s = lax.axis_index("s")
        pltpu.sync_copy(z_hbm, acc.at[pl.ds(s * stripe, stripe)])
        plsc.subcore_barrier()
        per_sc = (nchunks - c + 1) // 2
        nj = (per_sc - s + SC_NS - 1) // SC_NS

        @pl.loop(0, nj)
        def _chunk(j):
            kk = 2 * (s + SC_NS * j) + c
            off = pl.multiple_of(kk * C, C)
            pltpu.sync_copy(dst_hbm.at[pl.ds(off, C)], idx)
            cp = pltpu.async_copy(sp_hbm.at[pl.ds(off, C)], rows, sem)
            cp.wait()
            pltpu.sync_copy(rows, acc.at[idx], add=True)

        plsc.subcore_barrier()
        pltpu.sync_copy(acc.at[pl.ds(s * stripe, stripe)],
                        out_hbm.at[c, pl.ds(s * stripe, stripe)])

    out = k(SP, dst, zeros)
    return out


_LG_BS = 8192
_LG_TRASH = 128


def _sc_scatter_lg(SP, sorted_eid, sorted_dst, starts, n):
    E = SP.shape[0]
    C = 128
    BS = _LG_BS
    assert E % C == 0
    nbins = -(-n // BS)
    assert nbins % SC_NC == 0
    passes = nbins // SC_NC
    stripe = (BS + _LG_TRASH) // SC_NS
    dump = BS // SC_NS
    zeros = jnp.zeros((stripe, 2 * HID), jnp.float32)
    mesh = plsc.VectorSubcoreMesh(core_axis_name="c", subcore_axis_name="s")

    @functools.partial(
        pl.kernel, mesh=mesh,
        out_type=jax.ShapeDtypeStruct((nbins * BS, 2 * HID), jnp.float32),
        scratch_types=[pltpu.VMEM((C,), jnp.int32),
                       pltpu.VMEM((C,), jnp.int32),
                       pltpu.VMEM((C,), jnp.int32),
                       pltpu.VMEM((C, 2 * HID), jnp.float32),
                       pltpu.VMEM_SHARED((BS + _LG_TRASH, 2 * HID),
                                         jnp.float32),
                       pltpu.VMEM((nbins, 16), jnp.int32),
                       pltpu.SemaphoreType.DMA],
    )
    def k(sp_hbm, eid_hbm, sdst_hbm, starts_hbm, z_hbm, out_hbm,
          idxd, idxe, locb, rows, acc, st_v, sem):
        c = lax.axis_index("c")
        s = lax.axis_index("s")
        pltpu.sync_copy(starts_hbm, st_v)
        for p in range(passes):
            b = p * SC_NC + c
            base_val = b * BS
            pltpu.sync_copy(z_hbm, acc.at[pl.ds(s * stripe, stripe)])
            plsc.subcore_barrier()
            row = st_v[b]
            lo = row[0]
            hi = row[1]
            c0 = lo // C
            c1 = (hi + C - 1) // C
            nj = jnp.maximum(0, (c1 - c0 - s + SC_NS - 1) // SC_NS)

            @pl.loop(0, nj)
            def _chunk(j):
                kk = c0 + s + SC_NS * j
                off = pl.multiple_of(kk * C, C)
                pltpu.sync_copy(sdst_hbm.at[pl.ds(off, C)], idxd)
                pltpu.sync_copy(eid_hbm.at[pl.ds(off, C)], idxe)
                cp = pltpu.async_copy(sp_hbm.at[idxe], rows, sem)
                cp.wait()
                for i in range(C // 16):
                    v = idxd[pl.ds(i * 16, 16)]
                    lv = v - base_val
                    valid = (lv >= 0) & (lv < BS)
                    trash = BS + i * 16 + lax.iota(jnp.int32, 16)
                    locb[pl.ds(i * 16, 16)] = jnp.where(valid, lv, trash)
                pltpu.sync_copy(rows, acc.at[locb], add=True)

            plsc.subcore_barrier()
            pltpu.sync_copy(acc.at[pl.ds(s * dump, dump)],
                            out_hbm.at[pl.ds(base_val + s * dump, dump)])
            plsc.subcore_barrier()

    return k(SP, sorted_eid, sorted_dst, starts, zeros)


def _egc_layer(x, y, W, b, src, dst, n, lg_sort=None):
    Wcat = jnp.concatenate([W[0], W[4], W[1], W[3]], axis=1)
    bcat = jnp.concatenate([b[0], b[4], b[1], b[3]])
    T_sg, T_dx = _mm3(x, Wcat, bcat)
    yW2 = _mm(y, W[2], b[2])
    G1, G2 = _gather_rows(T_sg, T_dx, src, dst)
    m, SP, stats_m = _edge_ew(G1, G2, yW2)
    if lg_sort is None:
        parts_arr = _sc_scatter_node(SP, dst, n)
        parts = [parts_arr[0], parts_arr[1]]
    else:
        sorted_eid, sorted_dst, starts = lg_sort
        parts = [_sc_scatter_lg(SP, sorted_eid, sorted_dst, starts, n)]
    t, stats_t = _add_div_stats(T_dx, parts)
    x_new = _residual_bnsilu(t, stats_t, x)
    y_new = _residual_bnsilu(m, stats_m, y)
    return x_new, y_new


def kernel(atom_features, r, angle_h, edge_index, lg_edge_index, W_atom,
           b_atom, W_e1, b_e1, W_e2, b_e2, W_a1, b_a1, W_a2, b_a2, egc_W,
           egc_b, W_fc, b_fc):
    src, dst = edge_index[0], edge_index[1]
    lsrc, ldst = lg_edge_index[0], lg_edge_index[1]
    N = atom_features.shape[0]
    E = r.shape[0]

    t, s = _rbf_mm_stats(angle_h.reshape(-1, 1), W_a1, b_a1, -1.0, 1.0, 40,
                         is_r=False)
    t, s = _bnsilu_mm_stats(t, s, W_a2, b_a2)
    z = _bnsilu(t, s)

    t, s = _mm_stats(atom_features, W_atom, b_atom)
    x = _bnsilu(t, s)

    t, s = _rbf_mm_stats(r, W_e1, b_e1, 0.0, 8.0, 80, is_r=True)
    t, s = _bnsilu_mm_stats(t, s, W_e2, b_e2)
    y = _bnsilu(t, s)

    ELG = ldst.shape[0]
    nbins = -(-E // _LG_BS)
    sorted_ldst, lperm = lax.sort_key_val(
        ldst, jnp.arange(ELG, dtype=jnp.int32))
    starts = jnp.searchsorted(
        sorted_ldst, jnp.arange(nbins + 1, dtype=jnp.int32) * _LG_BS
    ).astype(jnp.int32)
    starts2 = jnp.zeros((nbins, 16), jnp.int32)
    starts2 = starts2.at[:, 0].set(starts[:-1]).at[:, 1].set(starts[1:])
    lg_sort = (lperm, sorted_ldst, starts2)

    x, m = _egc_layer(x, y, egc_W[0], egc_b[0], src, dst, N)
    y, z = _egc_layer(m, z, egc_W[1], egc_b[1], lsrc, ldst, E, lg_sort)
    x, m = _egc_layer(x, y, egc_W[2], egc_b[2], src, dst, N)
    y, z = _egc_layer(m, z, egc_W[3], egc_b[3], lsrc, ldst, E, lg_sort)
    x, y = _egc_layer(x, y, egc_W[4], egc_b[4], src, dst, N)
    x, y = _egc_layer(x, y, egc_W[5], egc_b[5], src, dst, N)

    return _readout(x, W_fc, b_fc)


# speedup vs baseline: 2.3015x; 1.0253x over previous
import functools

import numpy as np
import jax
import jax.numpy as jnp
from jax import lax
from jax.experimental import pallas as pl
from jax.experimental.pallas import tpu as pltpu
from jax.experimental.pallas import tpu_sc as plsc

SC_NC = 2
SC_NS = 16
SC_NW = SC_NC * SC_NS

HID = 64
BN_EPS = 1e-5
SEG_EPS = 1e-6


def _row_block(n, cap=2048):
    r = 8
    for c in range(8, cap + 1, 8):
        if n % c == 0:
            r = c
    return r


def _grid_call(kern, n, ins, in_widths, out_widths, n_stats, row_block=None):
    R = row_block or _row_block(n)
    grid = n // R
    in_specs = []
    for a, w in zip(ins, in_widths):
        if w is None:
            in_specs.append(
                pl.BlockSpec(a.shape, lambda i, nd=a.ndim: (0,) * nd))
        else:
            in_specs.append(pl.BlockSpec((R, w), lambda i: (i, 0)))
    out_specs = [pl.BlockSpec((R, w), lambda i: (i, 0)) for w in out_widths]
    out_shape = [jax.ShapeDtypeStruct((n, w), jnp.float32) for w in out_widths]
    for _ in range(n_stats):
        out_specs.append(pl.BlockSpec((2, HID), lambda i: (0, 0)))
        out_shape.append(jax.ShapeDtypeStruct((2, HID), jnp.float32))
    outs = pl.pallas_call(
        functools.partial(kern, grid=grid),
        grid=(grid,),
        in_specs=in_specs,
        out_specs=out_specs,
        out_shape=out_shape,
        scratch_shapes=[pltpu.VMEM((2, HID), jnp.float32)] * n_stats,
    )(*ins)
    return outs


def _accum_stats(t, i, grid, s_ref, acc_ref):
    ps = jnp.concatenate(
        [jnp.sum(t, axis=0, keepdims=True),
         jnp.sum(t * t, axis=0, keepdims=True)], axis=0)

    @pl.when(i == 0)
    def _():
        acc_ref[...] = ps

    @pl.when(i > 0)
    def _():
        acc_ref[...] = acc_ref[...] + ps

    @pl.when(i == grid - 1)
    def _():
        s_ref[...] = acc_ref[...]


def _bn_apply(t, s, n):
    mu = s[0:1, :] / n
    var = s[1:2, :] / n - mu * mu
    return (t - mu) * jax.lax.rsqrt(var + BN_EPS)


def _silu(v):
    return v * jax.nn.sigmoid(v)


def _mm_stats(x, W, b):
    n = x.shape[0]

    def kern(x_ref, w_ref, b_ref, t_ref, s_ref, acc_ref, *, grid):
        i = pl.program_id(0)
        t = jnp.dot(x_ref[...], w_ref[...],
                    preferred_element_type=jnp.float32) + b_ref[...]
        t_ref[...] = t
        _accum_stats(t, i, grid, s_ref, acc_ref)

    t, s = _grid_call(kern, n, [x, W, b.reshape(1, -1)],
                      [x.shape[1], None, None], [HID], 1)
    return t, s


def _rbf_mm_stats(d2col, W, b, vmin, vmax, bins, is_r):
    n = d2col.shape[0]
    centers = jnp.asarray(
        np.linspace(vmin, vmax, bins, dtype=np.float32)).reshape(1, bins)
    gamma = 1.0 / float(np.diff(np.linspace(vmin, vmax, bins)).mean())

    def kern(d_ref, c_ref, w_ref, b_ref, t_ref, s_ref, acc_ref, *, grid):
        i = pl.program_id(0)
        db = d_ref[...]
        if is_r:
            db = jnp.sqrt(jnp.sum(db * db, axis=1, keepdims=True))
        rbf = jnp.exp(-gamma * (db - c_ref[...]) ** 2)
        t = jnp.dot(rbf, w_ref[...],
                    preferred_element_type=jnp.float32) + b_ref[...]
        t_ref[...] = t
        _accum_stats(t, i, grid, s_ref, acc_ref)

    t, s = _grid_call(kern, n, [d2col, centers, W, b.reshape(1, -1)],
                      [d2col.shape[1], None, None, None], [HID], 1)
    return t, s


def _bnsilu_mm_stats(t1, s1, W, b):
    n = t1.shape[0]

    def kern(t1_ref, s1_ref, w_ref, b_ref, t_ref, s_ref, acc_ref, *, grid):
        i = pl.program_id(0)
        u = _silu(_bn_apply(t1_ref[...], s1_ref[...], n))
        t = jnp.dot(u, w_ref[...],
                    preferred_element_type=jnp.float32) + b_ref[...]
        t_ref[...] = t
        _accum_stats(t, i, grid, s_ref, acc_ref)

    t, s = _grid_call(kern, n, [t1, s1, W, b.reshape(1, -1)],
                      [HID, None, None, None], [HID], 1)
    return t, s


def _bnsilu(t, s):
    n = t.shape[0]

    def kern(t_ref, s_ref, o_ref, *, grid):
        o_ref[...] = _silu(_bn_apply(t_ref[...], s_ref[...], n))

    (o,) = _grid_call(kern, n, [t, s], [HID, None], [HID], 0)
    return o


def _residual_bnsilu(t, s, res):
    n = t.shape[0]

    def kern(t_ref, s_ref, r_ref, o_ref, *, grid):
        o_ref[...] = r_ref[...] + _silu(_bn_apply(t_ref[...], s_ref[...], n))

    (o,) = _grid_call(kern, n, [t, s, res], [HID, None, HID], [HID], 0)
    return o


def _mm3(x, Wcat, bcat):
    n = x.shape[0]

    def kern(x_ref, w_ref, b_ref, o1_ref, o2_ref, *, grid):
        t = jnp.dot(x_ref[...], w_ref[...],
                    preferred_element_type=jnp.float32) + b_ref[...]
        o1_ref[...] = t[:, :2 * HID]
        o2_ref[...] = t[:, 2 * HID:]

    o1, o2 = _grid_call(kern, n, [x, Wcat, bcat.reshape(1, -1)],
                        [HID, None, None], [2 * HID, 2 * HID], 0)
    return o1, o2


def _mm(x, W, b):
    n = x.shape[0]

    def kern(x_ref, w_ref, b_ref, o_ref, *, grid):
        o_ref[...] = jnp.dot(x_ref[...], w_ref[...],
                             preferred_element_type=jnp.float32) + b_ref[...]

    (o,) = _grid_call(kern, n, [x, W, b.reshape(1, -1)],
                      [HID, None, None], [HID], 0)
    return o


def _edge_ew(G1, G2, yW2):
    E = G1.shape[0]

    def kern(g1_ref, g2_ref, y_ref, m_ref, sp_ref, s_ref, acc_ref, *, grid):
        i = pl.program_id(0)
        g1 = g1_ref[...]
        m = g1[:, :HID] + g2_ref[:, :HID] + y_ref[...]
        sig = jax.nn.sigmoid(m)
        m_ref[...] = m
        sp_ref[:, :HID] = sig
        sp_ref[:, HID:] = g1[:, HID:] * sig
        _accum_stats(m, i, grid, s_ref, acc_ref)

    m, sp, s = _grid_call(kern, E, [G1, G2, yW2],
                          [2 * HID, 2 * HID, HID], [HID, 2 * HID], 1)
    return m, sp, s


def _add_div_stats(T_dx, parts):
    n = T_dx.shape[0]

    def kern(x_ref, *refs, grid):
        part_refs = refs[:len(parts)]
        t_ref, s_ref, acc_ref = refs[len(parts):]
        i = pl.program_id(0)
        ss = part_refs[0][...]
        for pr in part_refs[1:]:
            ss = ss + pr[...]
        t = x_ref[:, HID:] + ss[:, HID:] / (ss[:, :HID] + SEG_EPS)
        t_ref[...] = t
        _accum_stats(t, i, grid, s_ref, acc_ref)

    t, s = _grid_call(kern, n, [T_dx] + list(parts),
                      [2 * HID] + [2 * HID] * len(parts), [HID], 1)
    return t, s


def _readout(x, W_fc, b_fc):
    n = x.shape[0]
    R = _row_block(n)
    grid = n // R

    def kern(x_ref, w_ref, b_ref, o_ref, acc_ref):
        i = pl.program_id(0)
        ps = jnp.sum(x_ref[...], axis=0, keepdims=True)

        @pl.when(i == 0)
        def _():
            acc_ref[...] = ps

        @pl.when(i > 0)
        def _():
            acc_ref[...] = acc_ref[...] + ps

        @pl.when(i == grid - 1)
        def _():
            h = acc_ref[...] / n
            o_ref[...] = jnp.dot(h, w_ref[...],
                                 preferred_element_type=jnp.float32) + b_ref[...]

    out = pl.pallas_call(
        kern,
        grid=(grid,),
        in_specs=[pl.BlockSpec((R, HID), lambda i: (i, 0)),
                  pl.BlockSpec((HID, 1), lambda i: (0, 0)),
                  pl.BlockSpec((1, 1), lambda i: (0, 0))],
        out_specs=pl.BlockSpec((1, 1), lambda i: (0, 0)),
        out_shape=jax.ShapeDtypeStruct((1, 1), jnp.float32),
        scratch_shapes=[pltpu.VMEM((1, HID), jnp.float32)],
    )(x, W_fc, b_fc.reshape(1, 1))
    return jnp.squeeze(out)


def _gather_rows(T_sg, T_dx, src, dst):
    E = src.shape[0]
    C = 128
    assert E % C == 0
    nchunks = E // C
    base_nj = nchunks // SC_NW
    extra = nchunks - base_nj * SC_NW
    src2 = src.reshape(nchunks, C)
    dst2 = dst.reshape(nchunks, C)
    mesh = plsc.VectorSubcoreMesh(core_axis_name="c", subcore_axis_name="s")

    @functools.partial(
        pl.kernel, mesh=mesh,
        out_type=[jax.ShapeDtypeStruct((E, 2 * HID), jnp.float32),
                  jax.ShapeDtypeStruct((E, 2 * HID), jnp.float32)],
        scratch_types=[pltpu.VMEM((2, C), jnp.int32),
                       pltpu.VMEM((2, C), jnp.int32),
                       pltpu.VMEM((2 * C, 2 * HID), jnp.float32),
                       pltpu.VMEM((2 * C, 2 * HID), jnp.float32),
                       pltpu.SemaphoreType.DMA,
                       pltpu.SemaphoreType.DMA,
                       pltpu.SemaphoreType.DMA],
    )
    def k(tsg_hbm, tdx_hbm, src_hbm, dst_hbm, g1_hbm, g2_hbm,
          isrc, idst, rows1, rows2, sem0, sem1, sem_w):
        wid = lax.axis_index("s") * SC_NC + lax.axis_index("c")
        start = wid * base_nj + jnp.minimum(wid, extra)
        nj = base_nj + jnp.where(wid < extra, 1, 0)

        @pl.loop(0, nj // 2)
        def _pair(j2):
            ka = start + 2 * j2
            kb = ka + 1
            ia = pltpu.async_copy(src_hbm.at[ka], isrc.at[0], sem0)
            da = pltpu.async_copy(dst_hbm.at[ka], idst.at[0], sem0)
            ib = pltpu.async_copy(src_hbm.at[kb], isrc.at[1], sem1)
            db = pltpu.async_copy(dst_hbm.at[kb], idst.at[1], sem1)
            ia.wait()
            da.wait()
            g1a = pltpu.async_copy(tsg_hbm.at[isrc.at[0]],
                                   rows1.at[pl.ds(0, C)], sem0)
            g2a = pltpu.async_copy(tdx_hbm.at[idst.at[0]],
                                   rows2.at[pl.ds(0, C)], sem0)
            ib.wait()
            db.wait()
            g1b = pltpu.async_copy(tsg_hbm.at[isrc.at[1]],
                                   rows1.at[pl.ds(C, C)], sem1)
            g2b = pltpu.async_copy(tdx_hbm.at[idst.at[1]],
                                   rows2.at[pl.ds(C, C)], sem1)
            g1a.wait()
            g2a.wait()
            w1a = pltpu.async_copy(rows1.at[pl.ds(0, C)],
                                   g1_hbm.at[pl.ds(ka * C, C)], sem_w)
            w2a = pltpu.async_copy(rows2.at[pl.ds(0, C)],
                                   g2_hbm.at[pl.ds(ka * C, C)], sem_w)
            g1b.wait()
            g2b.wait()
            w1b = pltpu.async_copy(rows1.at[pl.ds(C, C)],
                                   g1_hbm.at[pl.ds(kb * C, C)], sem_w)
            w2b = pltpu.async_copy(rows2.at[pl.ds(C, C)],
                                   g2_hbm.at[pl.ds(kb * C, C)], sem_w)
            w1a.wait()
            w2a.wait()
            w1b.wait()
            w2b.wait()

        @pl.when(nj % 2 == 1)
        def _tail():
            kk = start + nj - 1
            pltpu.sync_copy(src_hbm.at[kk], isrc.at[0])
            pltpu.sync_copy(dst_hbm.at[kk], idst.at[0])
            ga = pltpu.async_copy(tsg_hbm.at[isrc.at[0]],
                                  rows1.at[pl.ds(0, C)], sem0)
            gb = pltpu.async_copy(tdx_hbm.at[idst.at[0]],
                                  rows2.at[pl.ds(0, C)], sem1)
            ga.wait()
            gb.wait()
            pltpu.sync_copy(rows1.at[pl.ds(0, C)],
                            g1_hbm.at[pl.ds(kk * C, C)])
            pltpu.sync_copy(rows2.at[pl.ds(0, C)],
                            g2_hbm.at[pl.ds(kk * C, C)])

    return k(T_sg, T_dx, src2, dst2)


def _segment_sum(SP, dst, n):
    return jax.ops.segment_sum(SP, dst, num_segments=n)


def _sc_scatter_node(SP, dst, n):
    E = SP.shape[0]
    C = 128
    assert E % C == 0
    nchunks = E // C
    n_pad = -(-n // (SC_NS * 8)) * (SC_NS * 8)
    stripe = n_pad // SC_NS
    zeros = jnp.zeros((stripe, 2 * HID), jnp.float32)
    dst2 = dst.reshape(nchunks, C)
    mesh = plsc.VectorSubcoreMesh(core_axis_name="c", subcore_axis_name="s")

    @functools.partial(
        pl.kernel, mesh=mesh,
        out_type=jax.ShapeDtypeStruct((SC_NC, n_pad, 2 * HID), jnp.float32),
        scratch_types=[pltpu.VMEM((2, C), jnp.int32),
                       pltpu.VMEM((2 * C, 2 * HID), jnp.float32),
                       pltpu.VMEM_SHARED((n_pad, 2 * HID), jnp.float32),
                       pltpu.SemaphoreType.DMA,
                       pltpu.SemaphoreType.DMA,
                       pltpu.SemaphoreType.DMA],
    )
    def k(sp_hbm, dst_hbm, z_hbm, out_hbm, idx, rows, acc, sem0, sem1,
          sem_s):
        c = lax.axis_index("c")
        s = lax.axis_index("s")
        pltpu.sync_copy(z_hbm, acc.at[pl.ds(s * stripe, stripe)])
        plsc.subcore_barrier()
        per_sc = (nchunks - c + 1) // 2
        nj = (per_sc - s + SC_NS - 1) // SC_NS

        @pl.loop(0, nj // 2)
        def _pair(j2):
            ka = 2 * (s + SC_NS * 2 * j2) + c
            kb = 2 * (s + SC_NS * (2 * j2 + 1)) + c
            ia = pltpu.async_copy(dst_hbm.at[ka], idx.at[0], sem0)
            ib = pltpu.async_copy(dst_hbm.at[kb], idx.at[1], sem1)
            ra = pltpu.async_copy(sp_hbm.at[pl.ds(ka * C, C)],
                                  rows.at[pl.ds(0, C)], sem0)
            rb = pltpu.async_copy(sp_hbm.at[pl.ds(kb * C, C)],
                                  rows.at[pl.ds(C, C)], sem1)
            ia.wait()
            ra.wait()
            sa = pltpu.async_copy(rows.at[pl.ds(0, C)], acc.at[idx.at[0]],
                                  sem_s, add=True)
            ib.wait()
            rb.wait()
            sb = pltpu.async_copy(rows.at[pl.ds(C, C)], acc.at[idx.at[1]],
                                  sem_s, add=True)
            sa.wait()
            sb.wait()

        @pl.when(nj % 2 == 1)
        def _tail():
            jj = nj - 1
            kk = 2 * (s + SC_NS * jj) + c
            pltpu.sync_copy(dst_hbm.at[kk], idx.at[0])
            cp = pltpu.async_copy(sp_hbm.at[pl.ds(kk * C, C)],
                                  rows.at[pl.ds(0, C)], sem0)
            cp.wait()
            pltpu.sync_copy(rows.at[pl.ds(0, C)], acc.at[idx.at[0]],
                            add=True)

        plsc.subcore_barrier()
        pltpu.sync_copy(acc.at[pl.ds(s * stripe, stripe)],
                        out_hbm.at[c, pl.ds(s * stripe, stripe)])

    out = k(SP, dst2, zeros)
    return out


_LG_BS = 8192
_LG_TRASH = 128


def _sc_scatter_lg(SP, sorted_eid, sorted_dst, starts, n):
    E = SP.shape[0]
    C = 128
    BS = _LG_BS
    assert E % C == 0
    nchunks = E // C
    nbins = -(-n // BS)
    assert nbins % SC_NC == 0
    passes = nbins // SC_NC
    stripe = (BS + _LG_TRASH) // SC_NS
    dump = BS // SC_NS
    zeros = jnp.zeros((stripe, 2 * HID), jnp.float32)
    eid2 = sorted_eid.reshape(nchunks, C)
    sdst2 = sorted_dst.reshape(nchunks, C)
    mesh = plsc.VectorSubcoreMesh(core_axis_name="c", subcore_axis_name="s")

    @functools.partial(
        pl.kernel, mesh=mesh,
        out_type=jax.ShapeDtypeStruct((nbins * BS, 2 * HID), jnp.float32),
        scratch_types=[pltpu.VMEM((2, C), jnp.int32),
                       pltpu.VMEM((2, C), jnp.int32),
                       pltpu.VMEM((2, C), jnp.int32),
                       pltpu.VMEM((2 * C, 2 * HID), jnp.float32),
                       pltpu.VMEM_SHARED((BS + _LG_TRASH, 2 * HID),
                                         jnp.float32),
                       pltpu.VMEM((nbins, 16), jnp.int32),
                       pltpu.SemaphoreType.DMA,
                       pltpu.SemaphoreType.DMA,
                       pltpu.SemaphoreType.DMA],
    )
    def k(sp_hbm, eid_hbm, sdst_hbm, starts_hbm, z_hbm, out_hbm,
          idxd, idxe, locb, rows, acc, st_v, sem0, sem1, sem_s):
        c = lax.axis_index("c")
        s = lax.axis_index("s")
        pltpu.sync_copy(starts_hbm, st_v)

        def compute_loc(buf, base_val):
            for i in range(C // 16):
                v = idxd[buf, pl.ds(i * 16, 16)]
                lv = v - base_val
                valid = (lv >= 0) & (lv < BS)
                trash = BS + i * 16 + lax.iota(jnp.int32, 16)
                locb[buf, pl.ds(i * 16, 16)] = jnp.where(valid, lv, trash)

        for p in range(passes):
            b = p * SC_NC + c
            base_val = b * BS
            pltpu.sync_copy(z_hbm, acc.at[pl.ds(s * stripe, stripe)])
            plsc.subcore_barrier()
            row = st_v[b]
            lo = row[0]
            hi = row[1]
            c0 = lo // C
            c1 = (hi + C - 1) // C
            nj = jnp.maximum(0, (c1 - c0 - s + SC_NS - 1) // SC_NS)

            @pl.loop(0, nj // 2)
            def _pair(j2):
                ka = c0 + s + SC_NS * 2 * j2
                kb = ka + SC_NS
                da = pltpu.async_copy(sdst_hbm.at[ka], idxd.at[0], sem0)
                ea = pltpu.async_copy(eid_hbm.at[ka], idxe.at[0], sem0)
                db = pltpu.async_copy(sdst_hbm.at[kb], idxd.at[1], sem1)
                eb = pltpu.async_copy(eid_hbm.at[kb], idxe.at[1], sem1)
                da.wait()
                ea.wait()
                ga = pltpu.async_copy(sp_hbm.at[idxe.at[0]],
                                      rows.at[pl.ds(0, C)], sem0)
                db.wait()
                eb.wait()
                gb = pltpu.async_copy(sp_hbm.at[idxe.at[1]],
                                      rows.at[pl.ds(C, C)], sem1)
                compute_loc(0, base_val)
                ga.wait()
                sa = pltpu.async_copy(rows.at[pl.ds(0, C)],
                                      acc.at[locb.at[0]], sem_s, add=True)
                compute_loc(1, base_val)
                gb.wait()
                sb = pltpu.async_copy(rows.at[pl.ds(C, C)],
                                      acc.at[locb.at[1]], sem_s, add=True)
                sa.wait()
                sb.wait()

            @pl.when(nj % 2 == 1)
            def _tail():
                kk = c0 + s + SC_NS * (nj - 1)
                pltpu.sync_copy(sdst_hbm.at[kk], idxd.at[0])
                pltpu.sync_copy(eid_hbm.at[kk], idxe.at[0])
                cp = pltpu.async_copy(sp_hbm.at[idxe.at[0]],
                                      rows.at[pl.ds(0, C)], sem0)
                compute_loc(0, base_val)
                cp.wait()
                pltpu.sync_copy(rows.at[pl.ds(0, C)], acc.at[locb.at[0]],
                                add=True)

            plsc.subcore_barrier()
            pltpu.sync_copy(acc.at[pl.ds(s * dump, dump)],
                            out_hbm.at[pl.ds(base_val + s * dump, dump)])
            plsc.subcore_barrier()

    return k(SP, eid2, sdst2, starts, zeros)


def _egc_layer(x, y, W, b, src, dst, n, lg_sort=None):
    Wcat = jnp.concatenate([W[0], W[4], W[1], W[3]], axis=1)
    bcat = jnp.concatenate([b[0], b[4], b[1], b[3]])
    T_sg, T_dx = _mm3(x, Wcat, bcat)
    yW2 = _mm(y, W[2], b[2])
    G1, G2 = _gather_rows(T_sg, T_dx, src, dst)
    m, SP, stats_m = _edge_ew(G1, G2, yW2)
    if lg_sort is None:
        parts_arr = _sc_scatter_node(SP, dst, n)
        parts = [parts_arr[0], parts_arr[1]]
    else:
        sorted_eid, sorted_dst, starts = lg_sort
        parts = [_sc_scatter_lg(SP, sorted_eid, sorted_dst, starts, n)]
    t, stats_t = _add_div_stats(T_dx, parts)
    x_new = _residual_bnsilu(t, stats_t, x)
    y_new = _residual_bnsilu(m, stats_m, y)
    return x_new, y_new


def kernel(atom_features, r, angle_h, edge_index, lg_edge_index, W_atom,
           b_atom, W_e1, b_e1, W_e2, b_e2, W_a1, b_a1, W_a2, b_a2, egc_W,
           egc_b, W_fc, b_fc):
    src, dst = edge_index[0], edge_index[1]
    lsrc, ldst = lg_edge_index[0], lg_edge_index[1]
    N = atom_features.shape[0]
    E = r.shape[0]

    t, s = _rbf_mm_stats(angle_h.reshape(-1, 1), W_a1, b_a1, -1.0, 1.0, 40,
                         is_r=False)
    t, s = _bnsilu_mm_stats(t, s, W_a2, b_a2)
    z = _bnsilu(t, s)

    t, s = _mm_stats(atom_features, W_atom, b_atom)
    x = _bnsilu(t, s)

    t, s = _rbf_mm_stats(r, W_e1, b_e1, 0.0, 8.0, 80, is_r=True)
    t, s = _bnsilu_mm_stats(t, s, W_e2, b_e2)
    y = _bnsilu(t, s)

    ELG = ldst.shape[0]
    nbins = -(-E // _LG_BS)
    sorted_ldst, lperm = lax.sort_key_val(
        ldst, jnp.arange(ELG, dtype=jnp.int32))
    starts = jnp.searchsorted(
        sorted_ldst, jnp.arange(nbins + 1, dtype=jnp.int32) * _LG_BS
    ).astype(jnp.int32)
    starts2 = jnp.zeros((nbins, 16), jnp.int32)
    starts2 = starts2.at[:, 0].set(starts[:-1]).at[:, 1].set(starts[1:])
    lg_sort = (lperm, sorted_ldst, starts2)

    x, m = _egc_layer(x, y, egc_W[0], egc_b[0], src, dst, N)
    y, z = _egc_layer(m, z, egc_W[1], egc_b[1], lsrc, ldst, E, lg_sort)
    x, m = _egc_layer(x, y, egc_W[2], egc_b[2], src, dst, N)
    y, z = _egc_layer(m, z, egc_W[3], egc_b[3], lsrc, ldst, E, lg_sort)
    x, y = _egc_layer(x, y, egc_W[4], egc_b[4], src, dst, N)
    x, y = _egc_layer(x, y, egc_W[5], egc_b[5], src, dst, N)

    return _readout(x, W_fc, b_fc)
